# Initial kernel scaffold; baseline (speedup 1.0000x reference)
import jax
import jax.numpy as jnp
from jax.experimental import pallas as pl


def kernel(x, logits):
    raise NotImplementedError("write your pallas kernel here")


# speedup vs baseline: 599.1760x; 599.1760x over previous
import functools

import jax
import jax.numpy as jnp
from jax import lax
from jax.experimental import pallas as pl
from jax.experimental.pallas import tpu as pltpu
from jax.experimental.pallas import tpu_sc as plsc

_B, _L, _A = 4096, 2048, 20
_NC, _NS, _LANES = 2, 16, 16
_NW = _NC * _NS
_ROWS_PER_W = _B // _NW
_GROUP = _LANES
_NGROUPS = _ROWS_PER_W // _GROUP
_GROUP_WORDS = _GROUP * _L


def _logsumexp_total_body(logits_ref, out_ref):
    lg = logits_ref[...]
    m = jnp.max(lg, axis=-1, keepdims=True)
    s = jnp.sum(jnp.exp(lg - m), axis=-1, keepdims=True)
    lse = m + jnp.log(s)
    out_ref[...] = jnp.sum(lse).reshape(1, 1)


_logsumexp_total = pl.pallas_call(
    _logsumexp_total_body,
    out_shape=jax.ShapeDtypeStruct((1, 1), jnp.float32),
)


def _gather_sum_body(x_hbm, table_hbm, out_hbm, table_v, xbuf0, xbuf1, out_v,
                     sem0, sem1, tsem):
    cid = lax.axis_index("c")
    sid = lax.axis_index("s")
    wid = sid * _NC + cid
    row0 = wid * _ROWS_PER_W

    tcopy = pltpu.async_copy(table_hbm, table_v, tsem)

    sems = [sem0, sem1]
    bufs = [xbuf0, xbuf1]

    def start_copy(g):
        return pltpu.async_copy(
            x_hbm.at[pl.ds((row0 + g * _GROUP) * _L, _GROUP_WORDS)],
            bufs[g % 2], sems[g % 2])

    pending = start_copy(0)
    rowoff = lax.iota(jnp.int32, _LANES) * _L
    tcopy.wait()

    for g in range(_NGROUPS):
        cur = pending
        if g + 1 < _NGROUPS:
            pending = start_copy(g + 1)
        cur.wait()
        xb = bufs[g % 2]

        @plsc.parallel_loop(0, _L, unroll=8,
                            carry=jnp.zeros((_LANES,), jnp.float32))
        def acc(l, a):
            xv = plsc.load_gather(xb, [rowoff + l])
            tv = plsc.load_gather(table_v, [xv + l * _A])
            return a + tv

        out_v[pl.ds(g * _GROUP, _GROUP)] = acc

    pltpu.sync_copy(out_v, out_hbm.at[pl.ds(row0, _ROWS_PER_W)])


_gather_sum = pl.kernel(
    _gather_sum_body,
    out_type=jax.ShapeDtypeStruct((_B,), jnp.float32),
    mesh=plsc.VectorSubcoreMesh(core_axis_name="c", subcore_axis_name="s"),
    compiler_params=pltpu.CompilerParams(needs_layout_passes=False),
    scratch_types=[
        pltpu.VMEM((_L * _A,), jnp.float32),
        pltpu.VMEM((_GROUP_WORDS,), jnp.int32),
        pltpu.VMEM((_GROUP_WORDS,), jnp.int32),
        pltpu.VMEM((_ROWS_PER_W,), jnp.float32),
        pltpu.SemaphoreType.DMA,
        pltpu.SemaphoreType.DMA,
        pltpu.SemaphoreType.DMA,
    ],
)


def kernel(x, logits):
    raw = _gather_sum(x.reshape(-1), logits.reshape(-1))
    c = _logsumexp_total(logits)
    return raw - c[0, 0]


# speedup vs baseline: 1243.8476x; 2.0759x over previous
import functools

import jax
import jax.numpy as jnp
from jax import lax
from jax.experimental import pallas as pl
from jax.experimental.pallas import tpu as pltpu
from jax.experimental.pallas import tpu_sc as plsc

_B, _L, _A = 4096, 2048, 20
_AP = 21
_NC, _NS, _LANES = 2, 16, 16
_NW = _NC * _NS
_ROWS_PER_W = _B // _NW
_GROUP = _LANES
_NGROUPS = _ROWS_PER_W // _GROUP
_GROUP_WORDS = _GROUP * _L
_CHUNKS = _L // _LANES


def _logsumexp_total_body(logits_ref, out_ref):
    lg = logits_ref[...]
    m = jnp.max(lg, axis=-1, keepdims=True)
    s = jnp.sum(jnp.exp(lg - m), axis=-1, keepdims=True)
    lse = m + jnp.log(s)
    out_ref[...] = jnp.sum(lse).reshape(1, 1)


_logsumexp_total = pl.pallas_call(
    _logsumexp_total_body,
    out_shape=jax.ShapeDtypeStruct((1, 1), jnp.float32),
)


def _gather_sum_body(x_hbm, table_hbm, out_hbm, table_v, xbuf0, xbuf1, out_v,
                     sem0, sem1, tsem):
    cid = lax.axis_index("c")
    sid = lax.axis_index("s")
    wid = sid * _NC + cid
    row0 = wid * _ROWS_PER_W

    tcopy = pltpu.async_copy(table_hbm, table_v, tsem)

    sems = [sem0, sem1]
    bufs = [xbuf0, xbuf1]

    def start_copy(g):
        return pltpu.async_copy(
            x_hbm.at[pl.ds((row0 + g * _GROUP) * _L, _GROUP_WORDS)],
            bufs[g % 2], sems[g % 2])

    pending = start_copy(0)
    lane = lax.iota(jnp.int32, _LANES)
    lane_off = lane * _AP
    zero = jnp.zeros((_LANES,), jnp.float32)
    tcopy.wait()

    for g in range(_NGROUPS):
        cur = pending
        if g + 1 < _NGROUPS:
            pending = start_copy(g + 1)
        cur.wait()
        xb = bufs[g % 2]

        def row_body(r, resvec, xb=xb):
            base = r * _L

            @plsc.parallel_loop(0, _CHUNKS, step=2, unroll=4,
                                carry=(zero, zero))
            def acc(j, carry):
                a0, a1 = carry
                x0 = xb[pl.ds(base + j * _LANES, _LANES)]
                x1 = xb[pl.ds(base + (j + 1) * _LANES, _LANES)]
                t0 = plsc.load_gather(
                    table_v, [x0 + lane_off + j * (_LANES * _AP)])
                t1 = plsc.load_gather(
                    table_v, [x1 + lane_off + (j + 1) * (_LANES * _AP)])
                return (a0 + t0, a1 + t1)

            rowsum = jnp.sum(acc[0] + acc[1])
            return jnp.where(lane == r, rowsum, resvec)

        out_v[pl.ds(g * _GROUP, _GROUP)] = lax.fori_loop(
            0, _GROUP, row_body, zero)

    pltpu.sync_copy(out_v, out_hbm.at[pl.ds(row0, _ROWS_PER_W)])


_gather_sum = pl.kernel(
    _gather_sum_body,
    out_type=jax.ShapeDtypeStruct((_B,), jnp.float32),
    mesh=plsc.VectorSubcoreMesh(core_axis_name="c", subcore_axis_name="s"),
    compiler_params=pltpu.CompilerParams(needs_layout_passes=False),
    scratch_types=[
        pltpu.VMEM((_L * _AP,), jnp.float32),
        pltpu.VMEM((_GROUP_WORDS,), jnp.int32),
        pltpu.VMEM((_GROUP_WORDS,), jnp.int32),
        pltpu.VMEM((_ROWS_PER_W,), jnp.float32),
        pltpu.SemaphoreType.DMA,
        pltpu.SemaphoreType.DMA,
        pltpu.SemaphoreType.DMA,
    ],
)


def kernel(x, logits):
    table = jnp.pad(logits, ((0, 0), (0, _AP - _A))).reshape(-1)
    raw = _gather_sum(x.reshape(-1), table)
    c = _logsumexp_total(logits)
    return raw - c[0, 0]


# speedup vs baseline: 1849.6659x; 1.4871x over previous
import functools

import jax
import jax.numpy as jnp
from jax import lax
from jax.experimental import pallas as pl
from jax.experimental.pallas import tpu as pltpu
from jax.experimental.pallas import tpu_sc as plsc

_B, _L, _A = 4096, 2048, 20
_AP = 21
_NC, _NS, _LANES = 2, 16, 16
_NW = _NC * _NS
_ROWS_PER_W = _B // _NW
_GROUP = _LANES
_NGROUPS = _ROWS_PER_W // _GROUP
_GROUP_WORDS = _GROUP * _L
_CHUNKS = _L // _LANES


def _logsumexp_total_body(logits_ref, out_ref):
    lg = logits_ref[...]
    m = jnp.max(lg, axis=-1, keepdims=True)
    s = jnp.sum(jnp.exp(lg - m), axis=-1, keepdims=True)
    lse = m + jnp.log(s)
    out_ref[...] = jnp.sum(lse).reshape(1, 1)


_logsumexp_total = pl.pallas_call(
    _logsumexp_total_body,
    out_shape=jax.ShapeDtypeStruct((1, 1), jnp.float32),
)


def _gather_sum_body(x_hbm, table_hbm, out_hbm, table_v, xbuf0, xbuf1, out_v,
                     sem0, sem1, tsem):
    cid = lax.axis_index("c")
    sid = lax.axis_index("s")
    wid = sid * _NC + cid
    row0 = wid * _ROWS_PER_W

    tcopy = pltpu.async_copy(table_hbm, table_v, tsem)

    sems = [sem0, sem1]
    bufs = [xbuf0, xbuf1]

    def start_copy(g):
        return pltpu.async_copy(
            x_hbm.at[pl.ds(row0 + g * _GROUP, _GROUP), :],
            bufs[g % 2], sems[g % 2])

    pending = start_copy(0)
    lane = lax.iota(jnp.int32, _LANES)
    lane_off = lane * _AP
    zero = jnp.zeros((_LANES,), jnp.float32)
    tcopy.wait()

    for g in range(_NGROUPS):
        cur = pending
        if g + 1 < _NGROUPS:
            pending = start_copy(g + 1)
        cur.wait()
        xb = bufs[g % 2]

        def row_body(r, resvec, xb=xb):

            @plsc.parallel_loop(0, _CHUNKS, step=2, unroll=4,
                                carry=(zero, zero))
            def acc(j, carry):
                a0, a1 = carry
                x0 = xb[r, pl.ds(j * _LANES, _LANES)]
                x1 = xb[r, pl.ds((j + 1) * _LANES, _LANES)]
                t0 = plsc.load_gather(
                    table_v, [x0 + lane_off + j * (_LANES * _AP)])
                t1 = plsc.load_gather(
                    table_v, [x1 + lane_off + (j + 1) * (_LANES * _AP)])
                return (a0 + t0, a1 + t1)

            rowsum = jnp.sum(acc[0] + acc[1])
            return jnp.where(lane == r, rowsum, resvec)

        out_v[pl.ds(g * _GROUP, _GROUP)] = lax.fori_loop(
            0, _GROUP, row_body, zero)

    pltpu.sync_copy(out_v, out_hbm.at[pl.ds(row0, _ROWS_PER_W)])


_gather_sum = pl.kernel(
    _gather_sum_body,
    out_type=jax.ShapeDtypeStruct((_B,), jnp.float32),
    mesh=plsc.VectorSubcoreMesh(core_axis_name="c", subcore_axis_name="s"),
    compiler_params=pltpu.CompilerParams(needs_layout_passes=False),
    scratch_types=[
        pltpu.VMEM((_L * _AP,), jnp.float32),
        pltpu.VMEM((_GROUP, _L), jnp.int32),
        pltpu.VMEM((_GROUP, _L), jnp.int32),
        pltpu.VMEM((_ROWS_PER_W,), jnp.float32),
        pltpu.SemaphoreType.DMA,
        pltpu.SemaphoreType.DMA,
        pltpu.SemaphoreType.DMA,
    ],
)


def kernel(x, logits):
    table = jnp.pad(logits, ((0, 0), (0, _AP - _A))).reshape(-1)
    raw = _gather_sum(x, table)
    c = _logsumexp_total(logits)
    return raw - c[0, 0]
